# baseline (device time: 121680 ns/iter reference)
import jax
import jax.numpy as jnp
from jax import lax
from jax.experimental import pallas as pl
from jax.experimental.pallas import tpu as pltpu

M = 2048
HALF = M // 2
QUAR = M // 4
CHALF = M // 2
K = 8192
KB = 512
NK = K // KB

_DOT_DIMS = (((1,), (1,)), ((), ()))


def kernel(dy, W):
    def body(dy_ref, w_ref, out_ref, a_buf, w_buf, acc, send_stage,
             recv_rs, recv_agx, recv_agy, recv_dgx, recv_dgy,
             load_sems, send_sems, recv_sems):
        my_x = lax.axis_index("x")
        my_y = lax.axis_index("y")
        x_nbr = (1 - my_x, my_y)
        y_nbr = (my_x, 1 - my_y)
        row0 = my_x * HALF

        barrier_sem = pltpu.get_barrier_semaphore()
        for nbr in (x_nbr, y_nbr):
            pl.semaphore_signal(
                barrier_sem, inc=1, device_id=nbr,
                device_id_type=pl.DeviceIdType.MESH,
            )

        def start_load(slot, kidx):
            a_cp = pltpu.make_async_copy(
                dy_ref.at[pl.ds(row0, HALF), pl.ds(kidx * KB, KB)],
                a_buf.at[slot],
                load_sems.at[0, slot],
            )
            w_cp = pltpu.make_async_copy(
                w_ref.at[:, pl.ds(kidx * KB, KB)],
                w_buf.at[slot],
                load_sems.at[1, slot],
            )
            a_cp.start()
            w_cp.start()
            return a_cp, w_cp

        pending = start_load(0, 0)
        for k in range(NK):
            slot = k % 2
            nxt = start_load(1 - slot, k + 1) if k + 1 < NK else None
            pending[0].wait()
            pending[1].wait()
            prod = lax.dot_general(
                a_buf[slot].astype(jnp.bfloat16),
                w_buf[slot].astype(jnp.bfloat16),
                _DOT_DIMS,
                preferred_element_type=jnp.float32,
            )
            if k == 0:
                acc[...] = prod
            else:
                acc[...] = acc[...] + prod
            pending = nxt

        loc_mine = pl.ds(my_y * QUAR, QUAR)
        loc_send = pl.ds((1 - my_y) * QUAR, QUAR)
        mine = pl.ds(my_x * HALF + my_y * QUAR, QUAR)
        from_x = pl.ds((1 - my_x) * HALF + my_y * QUAR, QUAR)
        from_y = pl.ds(my_x * HALF + (1 - my_y) * QUAR, QUAR)
        diag = pl.ds((1 - my_x) * HALF + (1 - my_y) * QUAR, QUAR)

        send_stage[...] = acc[loc_send, :].astype(jnp.bfloat16)
        pl.semaphore_wait(barrier_sem, 2)

        rdma1 = pltpu.make_async_remote_copy(
            src_ref=send_stage,
            dst_ref=recv_rs,
            send_sem=send_sems.at[0],
            recv_sem=recv_sems.at[0],
            device_id=y_nbr,
            device_id_type=pl.DeviceIdType.MESH,
        )
        rdma1.start()
        rdma1.wait()
        out_ref[mine, :] = (
            acc[loc_mine, :] + recv_rs[...].astype(jnp.float32)
        ).astype(jnp.bfloat16)

        rdma2 = pltpu.make_async_remote_copy(
            src_ref=out_ref.at[mine, :],
            dst_ref=recv_agx,
            send_sem=send_sems.at[1],
            recv_sem=recv_sems.at[1],
            device_id=x_nbr,
            device_id_type=pl.DeviceIdType.MESH,
        )
        rdma3 = pltpu.make_async_remote_copy(
            src_ref=out_ref.at[mine, :],
            dst_ref=recv_agy,
            send_sem=send_sems.at[2],
            recv_sem=recv_sems.at[2],
            device_id=y_nbr,
            device_id_type=pl.DeviceIdType.MESH,
        )
        rdma2.start()
        rdma3.start()
        rdma2.wait()
        rdma3.wait()
        out_ref[from_x, :] = recv_agx[...]
        out_ref[from_y, :] = recv_agy[...]

        rdma4a = pltpu.make_async_remote_copy(
            src_ref=out_ref.at[from_y, pl.ds(0, CHALF)],
            dst_ref=recv_dgx,
            send_sem=send_sems.at[3],
            recv_sem=recv_sems.at[3],
            device_id=x_nbr,
            device_id_type=pl.DeviceIdType.MESH,
        )
        rdma4b = pltpu.make_async_remote_copy(
            src_ref=out_ref.at[from_x, pl.ds(CHALF, CHALF)],
            dst_ref=recv_dgy,
            send_sem=send_sems.at[4],
            recv_sem=recv_sems.at[4],
            device_id=y_nbr,
            device_id_type=pl.DeviceIdType.MESH,
        )
        rdma4a.start()
        rdma4b.start()
        rdma4a.wait()
        rdma4b.wait()
        out_ref[diag, pl.ds(0, CHALF)] = recv_dgx[...]
        out_ref[diag, pl.ds(CHALF, CHALF)] = recv_dgy[...]

    return pl.pallas_call(
        body,
        out_shape=jax.ShapeDtypeStruct((M, M), jnp.bfloat16),
        in_specs=[
            pl.BlockSpec(memory_space=pl.ANY),
            pl.BlockSpec(memory_space=pl.ANY),
        ],
        out_specs=pl.BlockSpec(memory_space=pltpu.VMEM),
        scratch_shapes=[
            pltpu.VMEM((2, HALF, KB), jnp.float32),
            pltpu.VMEM((2, M, KB), jnp.float32),
            pltpu.VMEM((HALF, M), jnp.float32),
            pltpu.VMEM((QUAR, M), jnp.bfloat16),
            pltpu.VMEM((QUAR, M), jnp.bfloat16),
            pltpu.VMEM((QUAR, M), jnp.bfloat16),
            pltpu.VMEM((QUAR, M), jnp.bfloat16),
            pltpu.VMEM((QUAR, CHALF), jnp.bfloat16),
            pltpu.VMEM((QUAR, CHALF), jnp.bfloat16),
            pltpu.SemaphoreType.DMA((2, 2)),
            pltpu.SemaphoreType.DMA((5,)),
            pltpu.SemaphoreType.DMA((5,)),
        ],
        compiler_params=pltpu.CompilerParams(
            collective_id=0, vmem_limit_bytes=64 * 1024 * 1024,
        ),
    )(dy, W)


# device time: 112657 ns/iter; 1.0801x vs baseline; 1.0801x over previous
import jax
import jax.numpy as jnp
from jax import lax
from jax.experimental import pallas as pl
from jax.experimental.pallas import tpu as pltpu

M = 2048
HALF = M // 2
QUAR = M // 4
CHALF = M // 2
K = 8192
KB = 1024
NK = K // KB

_DOT_DIMS = (((1,), (1,)), ((), ()))


def kernel(dy, W):
    def body(dy_ref, w_ref, out_ref, a_buf, w_buf, acc_send, acc_mine,
             send_stage, recv_rs, recv_agx, recv_agy, recv_dgx, recv_dgy,
             load_sems, send_sems, recv_sems):
        my_x = lax.axis_index("x")
        my_y = lax.axis_index("y")
        x_nbr = (1 - my_x, my_y)
        y_nbr = (my_x, 1 - my_y)

        barrier_sem = pltpu.get_barrier_semaphore()
        for nbr in (x_nbr, y_nbr):
            pl.semaphore_signal(
                barrier_sem, inc=1, device_id=nbr,
                device_id_type=pl.DeviceIdType.MESH,
            )

        def gemm_quarter(row0, acc):
            def start_load(slot, kidx):
                a_cp = pltpu.make_async_copy(
                    dy_ref.at[pl.ds(row0, QUAR), pl.ds(kidx * KB, KB)],
                    a_buf.at[slot],
                    load_sems.at[0, slot],
                )
                w_cp = pltpu.make_async_copy(
                    w_ref.at[:, pl.ds(kidx * KB, KB)],
                    w_buf.at[slot],
                    load_sems.at[1, slot],
                )
                a_cp.start()
                w_cp.start()
                return a_cp, w_cp

            pending = start_load(0, 0)
            for k in range(NK):
                slot = k % 2
                nxt = start_load(1 - slot, k + 1) if k + 1 < NK else None
                pending[0].wait()
                pending[1].wait()
                prod = lax.dot_general(
                    a_buf[slot].astype(jnp.bfloat16),
                    w_buf[slot].astype(jnp.bfloat16),
                    _DOT_DIMS,
                    preferred_element_type=jnp.float32,
                )
                if k == 0:
                    acc[...] = prod
                else:
                    acc[...] = acc[...] + prod
                pending = nxt

        gemm_quarter(my_x * HALF + (1 - my_y) * QUAR, acc_send)
        send_stage[...] = acc_send[...].astype(jnp.bfloat16)
        pl.semaphore_wait(barrier_sem, 2)

        rdma1 = pltpu.make_async_remote_copy(
            src_ref=send_stage,
            dst_ref=recv_rs,
            send_sem=send_sems.at[0],
            recv_sem=recv_sems.at[0],
            device_id=y_nbr,
            device_id_type=pl.DeviceIdType.MESH,
        )
        rdma1.start()

        gemm_quarter(my_x * HALF + my_y * QUAR, acc_mine)

        mine = pl.ds(my_x * HALF + my_y * QUAR, QUAR)
        from_x = pl.ds((1 - my_x) * HALF + my_y * QUAR, QUAR)
        from_y = pl.ds(my_x * HALF + (1 - my_y) * QUAR, QUAR)
        diag = pl.ds((1 - my_x) * HALF + (1 - my_y) * QUAR, QUAR)

        rdma1.wait()
        out_ref[mine, :] = (
            acc_mine[...] + recv_rs[...].astype(jnp.float32)
        ).astype(jnp.bfloat16)

        rdma2 = pltpu.make_async_remote_copy(
            src_ref=out_ref.at[mine, :],
            dst_ref=recv_agx,
            send_sem=send_sems.at[1],
            recv_sem=recv_sems.at[1],
            device_id=x_nbr,
            device_id_type=pl.DeviceIdType.MESH,
        )
        rdma3 = pltpu.make_async_remote_copy(
            src_ref=out_ref.at[mine, :],
            dst_ref=recv_agy,
            send_sem=send_sems.at[2],
            recv_sem=recv_sems.at[2],
            device_id=y_nbr,
            device_id_type=pl.DeviceIdType.MESH,
        )
        rdma2.start()
        rdma3.start()
        rdma2.wait()
        rdma3.wait()
        out_ref[from_x, :] = recv_agx[...]
        out_ref[from_y, :] = recv_agy[...]

        rdma4a = pltpu.make_async_remote_copy(
            src_ref=out_ref.at[from_y, pl.ds(0, CHALF)],
            dst_ref=recv_dgx,
            send_sem=send_sems.at[3],
            recv_sem=recv_sems.at[3],
            device_id=x_nbr,
            device_id_type=pl.DeviceIdType.MESH,
        )
        rdma4b = pltpu.make_async_remote_copy(
            src_ref=out_ref.at[from_x, pl.ds(CHALF, CHALF)],
            dst_ref=recv_dgy,
            send_sem=send_sems.at[4],
            recv_sem=recv_sems.at[4],
            device_id=y_nbr,
            device_id_type=pl.DeviceIdType.MESH,
        )
        rdma4a.start()
        rdma4b.start()
        rdma4a.wait()
        rdma4b.wait()
        out_ref[diag, pl.ds(0, CHALF)] = recv_dgx[...]
        out_ref[diag, pl.ds(CHALF, CHALF)] = recv_dgy[...]

    return pl.pallas_call(
        body,
        out_shape=jax.ShapeDtypeStruct((M, M), jnp.bfloat16),
        in_specs=[
            pl.BlockSpec(memory_space=pl.ANY),
            pl.BlockSpec(memory_space=pl.ANY),
        ],
        out_specs=pl.BlockSpec(memory_space=pltpu.VMEM),
        scratch_shapes=[
            pltpu.VMEM((2, QUAR, KB), jnp.float32),
            pltpu.VMEM((2, M, KB), jnp.float32),
            pltpu.VMEM((QUAR, M), jnp.float32),
            pltpu.VMEM((QUAR, M), jnp.float32),
            pltpu.VMEM((QUAR, M), jnp.bfloat16),
            pltpu.VMEM((QUAR, M), jnp.bfloat16),
            pltpu.VMEM((QUAR, M), jnp.bfloat16),
            pltpu.VMEM((QUAR, M), jnp.bfloat16),
            pltpu.VMEM((QUAR, CHALF), jnp.bfloat16),
            pltpu.VMEM((QUAR, CHALF), jnp.bfloat16),
            pltpu.SemaphoreType.DMA((2, 2)),
            pltpu.SemaphoreType.DMA((5,)),
            pltpu.SemaphoreType.DMA((5,)),
        ],
        compiler_params=pltpu.CompilerParams(
            collective_id=0, vmem_limit_bytes=64 * 1024 * 1024,
        ),
    )(dy, W)


# device time: 97082 ns/iter; 1.2534x vs baseline; 1.1604x over previous
import jax
import jax.numpy as jnp
from jax import lax
from jax.experimental import pallas as pl
from jax.experimental.pallas import tpu as pltpu

M = 2048
HALF = M // 2
QUAR = M // 4
K = 8192
NCH = 4
CB = M // NCH
KT = 2048
NKT = K // KT

_DOT_DIMS = (((1,), (1,)), ((), ()))

RS, AGX, AGY, DG = 0, 1, 2, 3


def kernel(dy, W):
    def body(dy_ref, w_ref, out_ref, a_bf16, a_stage, w_buf, acc,
             rs_stage, recv_rs, load_sems, send_sems, recv_sems):
        my_x = lax.axis_index("x")
        my_y = lax.axis_index("y")
        x_nbr = (1 - my_x, my_y)
        y_nbr = (my_x, 1 - my_y)
        row0 = my_x * HALF

        barrier_sem = pltpu.get_barrier_semaphore()
        for nbr in (x_nbr, y_nbr):
            pl.semaphore_signal(
                barrier_sem, inc=1, device_id=nbr,
                device_id_type=pl.DeviceIdType.MESH,
            )

        loc_mine = pl.ds(my_y * QUAR, QUAR)
        loc_send = pl.ds((1 - my_y) * QUAR, QUAR)
        mine = pl.ds(my_x * HALF + my_y * QUAR, QUAR)
        from_y = pl.ds(my_x * HALF + (1 - my_y) * QUAR, QUAR)

        def start_a(kt):
            cp = pltpu.make_async_copy(
                dy_ref.at[pl.ds(row0, HALF), pl.ds(kt * KT, KT)],
                a_stage,
                load_sems.at[0],
            )
            cp.start()
            return cp

        def start_w(t):
            c, kt = divmod(t, NKT)
            cp = pltpu.make_async_copy(
                w_ref.at[pl.ds(c * CB, CB), pl.ds(kt * KT, KT)],
                w_buf.at[t % 2],
                load_sems.at[1 + t % 2],
            )
            cp.start()
            return cp

        def remote(src, dst, stage, c, target):
            return pltpu.make_async_remote_copy(
                src_ref=src,
                dst_ref=dst,
                send_sem=send_sems.at[stage, c],
                recv_sem=recv_sems.at[stage, c],
                device_id=target,
                device_id_type=pl.DeviceIdType.MESH,
            )

        rs = [None] * NCH
        agx = [None] * NCH
        agy = [None] * NCH
        dg = [None] * NCH

        def finish_chunk(j):
            cols = pl.ds(j * CB, CB)
            rs[j].wait()
            out_ref[mine, cols] = (
                acc[j % 2, loc_mine, :]
                + recv_rs[:, cols].astype(jnp.float32)
            ).astype(jnp.bfloat16)
            agx[j] = remote(out_ref.at[mine, cols], out_ref.at[mine, cols],
                            AGX, j, x_nbr)
            agy[j] = remote(out_ref.at[mine, cols], out_ref.at[mine, cols],
                            AGY, j, y_nbr)
            agx[j].start()
            agy[j].start()

        def relay_diag(j):
            cols = pl.ds(j * CB, CB)
            agy[j].wait_recv()
            dg[j] = remote(out_ref.at[from_y, cols], out_ref.at[from_y, cols],
                           DG, j, x_nbr)
            dg[j].start()

        a_cp = start_a(0)
        w_cp = [start_w(0), None]
        for c in range(NCH):
            for kt in range(NKT):
                t = c * NKT + kt
                slot = t % 2
                if c == 0:
                    a_cp.wait()
                    a_bf16[:, pl.ds(kt * KT, KT)] = (
                        a_stage[...].astype(jnp.bfloat16)
                    )
                    if kt + 1 < NKT:
                        a_cp = start_a(kt + 1)
                if t + 1 < NCH * NKT:
                    w_cp[1 - slot] = start_w(t + 1)
                w_cp[slot].wait()
                prod = lax.dot_general(
                    a_bf16[:, pl.ds(kt * KT, KT)],
                    w_buf[slot].astype(jnp.bfloat16),
                    _DOT_DIMS,
                    preferred_element_type=jnp.float32,
                )
                if kt == 0:
                    acc[c % 2, :, :] = prod
                else:
                    acc[c % 2, :, :] = acc[c % 2, :, :] + prod

            rs_stage[c % 2, :, :] = acc[c % 2, loc_send, :].astype(jnp.bfloat16)
            if c == 0:
                pl.semaphore_wait(barrier_sem, 2)
            rs[c] = remote(rs_stage.at[c % 2],
                           recv_rs.at[:, pl.ds(c * CB, CB)], RS, c, y_nbr)
            rs[c].start()
            if c >= 1:
                finish_chunk(c - 1)
            if c >= 2:
                relay_diag(c - 2)

        finish_chunk(NCH - 1)
        relay_diag(NCH - 2)
        relay_diag(NCH - 1)
        for j in range(NCH):
            agx[j].wait()
            agy[j].wait_send()
            dg[j].wait()

    return pl.pallas_call(
        body,
        out_shape=jax.ShapeDtypeStruct((M, M), jnp.bfloat16),
        in_specs=[
            pl.BlockSpec(memory_space=pl.ANY),
            pl.BlockSpec(memory_space=pl.ANY),
        ],
        out_specs=pl.BlockSpec(memory_space=pltpu.VMEM),
        scratch_shapes=[
            pltpu.VMEM((HALF, K), jnp.bfloat16),
            pltpu.VMEM((HALF, KT), jnp.float32),
            pltpu.VMEM((2, CB, KT), jnp.float32),
            pltpu.VMEM((2, HALF, CB), jnp.float32),
            pltpu.VMEM((2, QUAR, CB), jnp.bfloat16),
            pltpu.VMEM((QUAR, M), jnp.bfloat16),
            pltpu.SemaphoreType.DMA((3,)),
            pltpu.SemaphoreType.DMA((4, NCH)),
            pltpu.SemaphoreType.DMA((4, NCH)),
        ],
        compiler_params=pltpu.CompilerParams(
            collective_id=0, vmem_limit_bytes=64 * 1024 * 1024,
        ),
    )(dy, W)
